# Initial kernel scaffold; baseline (speedup 1.0000x reference)
import jax
import jax.numpy as jnp
from jax.experimental import pallas as pl


def kernel(x, edge_index, edge_weight, W1, b1, ln_gamma, ln_beta, W2, b2):
    raise NotImplementedError("write your pallas kernel here")


# speedup vs baseline: 9.3990x; 9.3990x over previous
import functools

import jax
import jax.numpy as jnp
from jax import lax
from jax.experimental import pallas as pl
from jax.experimental.pallas import tpu as pltpu
from jax.experimental.pallas import tpu_sc as plsc

N = 10000
E = 320000
D = 128
P = 10240
NC = 2
NS = 16
E_PER_SC = E // NC
E_PER_TILE = E // (NC * NS)
WIN = 128
NWIN = E_PER_TILE // WIN
TAIL = E_PER_TILE - NWIN * WIN
ROWS_PER_TILE = P // NS

_mesh = plsc.VectorSubcoreMesh(core_axis_name="c", subcore_axis_name="s")
_sc_params = pltpu.CompilerParams(needs_layout_passes=False)


def _k1_body(dst_hbm, w_hbm, out_hbm, deg_sp, dstw, ww, dstt, wt, zero_v):
    c = lax.axis_index("c")
    s = lax.axis_index("s")

    @pl.loop(0, 40)
    def _zero(j):
        zero_v[pl.ds(j * 16, 16)] = jnp.zeros((16,), jnp.float32)

    pltpu.sync_copy(zero_v, deg_sp.at[pl.ds(s * ROWS_PER_TILE, ROWS_PER_TILE)])
    plsc.subcore_barrier()

    base = c * E_PER_SC + s * E_PER_TILE

    @pl.loop(0, NWIN)
    def _win(i):
        off = base + i * WIN
        pltpu.sync_copy(dst_hbm.at[pl.ds(off, WIN)], dstw)
        pltpu.sync_copy(w_hbm.at[pl.ds(off, WIN)], ww)
        pltpu.sync_copy(ww, deg_sp.at[dstw], add=True)

    off = base + NWIN * WIN
    pltpu.sync_copy(dst_hbm.at[pl.ds(off, TAIL)], dstt)
    pltpu.sync_copy(w_hbm.at[pl.ds(off, TAIL)], wt)
    pltpu.sync_copy(wt, deg_sp.at[dstt], add=True)

    plsc.subcore_barrier()
    r0 = s * ROWS_PER_TILE
    pltpu.sync_copy(deg_sp.at[pl.ds(r0, ROWS_PER_TILE)],
                    out_hbm.at[c, pl.ds(r0, ROWS_PER_TILE)])


_k1 = pl.kernel(
    _k1_body,
    out_type=jax.ShapeDtypeStruct((NC, P), jnp.float32),
    mesh=_mesh,
    compiler_params=_sc_params,
    scratch_types=[
        pltpu.VMEM_SHARED((P,), jnp.float32),
        pltpu.VMEM((WIN,), jnp.int32),
        pltpu.VMEM((WIN,), jnp.float32),
        pltpu.VMEM((TAIL,), jnp.int32),
        pltpu.VMEM((TAIL,), jnp.float32),
        pltpu.VMEM((ROWS_PER_TILE,), jnp.float32),
    ],
)


def _scale_rows(rows, cw, nrows):
    @pl.loop(0, nrows)
    def _row(j):
        idx = jnp.broadcast_to(j, (16,)).astype(jnp.int32)
        cb = plsc.load_gather(cw, [idx])
        for f in range(D // 16):
            sl = pl.ds(f * 16, 16)
            rows[j, sl] = rows[j, sl] * cb


def _coeffs(srcw, ww, cw, dinv_v, n):
    for k in range(n // 16):
        sl = pl.ds(k * 16, 16)
        dv = plsc.load_gather(dinv_v, [srcw[sl]])
        cw[sl] = ww[sl] * dv


def _k3_body(h_hbm, src_hbm, dst_hbm, w_hbm, dinv_hbm, out_hbm,
             acc_sp, dinv_v, srcw, dstw, ww, cw, rows,
             srct, dstt, wt, ct, rowst, zero_v, sem):
    c = lax.axis_index("c")
    s = lax.axis_index("s")

    @pl.loop(0, 128)
    def _zero(j):
        for f in range(D // 16):
            zero_v[j, pl.ds(f * 16, 16)] = jnp.zeros((16,), jnp.float32)

    for t in range(ROWS_PER_TILE // 128):
        pltpu.sync_copy(zero_v, acc_sp.at[pl.ds(s * ROWS_PER_TILE + t * 128, 128)])
    pltpu.sync_copy(dinv_hbm, dinv_v)
    plsc.subcore_barrier()

    base = c * E_PER_SC + s * E_PER_TILE

    @pl.loop(0, NWIN)
    def _win(i):
        off = base + i * WIN
        pltpu.sync_copy(src_hbm.at[pl.ds(off, WIN)], srcw)
        pltpu.sync_copy(dst_hbm.at[pl.ds(off, WIN)], dstw)
        pltpu.sync_copy(w_hbm.at[pl.ds(off, WIN)], ww)
        pltpu.async_copy(h_hbm.at[srcw], rows, sem).wait()
        _coeffs(srcw, ww, cw, dinv_v, WIN)
        _scale_rows(rows, cw, WIN)
        pltpu.sync_copy(rows, acc_sp.at[dstw], add=True)

    off = base + NWIN * WIN
    pltpu.sync_copy(src_hbm.at[pl.ds(off, TAIL)], srct)
    pltpu.sync_copy(dst_hbm.at[pl.ds(off, TAIL)], dstt)
    pltpu.sync_copy(w_hbm.at[pl.ds(off, TAIL)], wt)
    pltpu.async_copy(h_hbm.at[srct], rowst, sem).wait()
    _coeffs(srct, wt, ct, dinv_v, TAIL)
    _scale_rows(rowst, ct, TAIL)
    pltpu.sync_copy(rowst, acc_sp.at[dstt], add=True)

    plsc.subcore_barrier()
    for t in range(ROWS_PER_TILE // 128):
        r0 = s * ROWS_PER_TILE + t * 128
        pltpu.sync_copy(acc_sp.at[pl.ds(r0, 128)], out_hbm.at[c, pl.ds(r0, 128)])


_k3 = pl.kernel(
    _k3_body,
    out_type=jax.ShapeDtypeStruct((NC, P, D), jnp.float32),
    mesh=_mesh,
    compiler_params=_sc_params,
    scratch_types=[
        pltpu.VMEM_SHARED((P, D), jnp.float32),
        pltpu.VMEM((P,), jnp.float32),
        pltpu.VMEM((WIN,), jnp.int32),
        pltpu.VMEM((WIN,), jnp.int32),
        pltpu.VMEM((WIN,), jnp.float32),
        pltpu.VMEM((WIN,), jnp.float32),
        pltpu.VMEM((WIN, D), jnp.float32),
        pltpu.VMEM((TAIL,), jnp.int32),
        pltpu.VMEM((TAIL,), jnp.int32),
        pltpu.VMEM((TAIL,), jnp.float32),
        pltpu.VMEM((TAIL,), jnp.float32),
        pltpu.VMEM((TAIL, D), jnp.float32),
        pltpu.VMEM((128, D), jnp.float32),
        pltpu.SemaphoreType.DMA,
    ],
)


def _k2_body(x_ref, w1_ref, degT_ref, h_ref, dinv_ref):
    h_ref[...] = jnp.dot(x_ref[...], w1_ref[...],
                         preferred_element_type=jnp.float32)
    d = degT_ref[:, 0:1] + degT_ref[:, 1:2] + 1.0
    dinv_ref[...] = lax.rsqrt(d)


def _k4_body(a0_ref, a1_ref, h1_ref, dinv_ref, b1_ref, g_ref, b_ref, w2_ref,
             h2_ref):
    d = dinv_ref[...]
    z = d * (a0_ref[...] + a1_ref[...] + d * h1_ref[...]) + b1_ref[...]
    z = jnp.maximum(z, 0.0)
    m = jnp.mean(z, axis=-1, keepdims=True)
    zc = z - m
    v = jnp.mean(zc * zc, axis=-1, keepdims=True)
    zn = g_ref[...] * zc * lax.rsqrt(v + 1e-5) + b_ref[...]
    h2_ref[...] = jnp.dot(zn, w2_ref[...], preferred_element_type=jnp.float32)


def _k5_body(a0_ref, a1_ref, h2_ref, dinv_ref, b2_ref, o_ref):
    d = dinv_ref[...]
    o_ref[...] = d * (a0_ref[...] + a1_ref[...] + d * h2_ref[...]) + b2_ref[...]


def _blk(shape, imap):
    return pl.BlockSpec(shape, imap)


_row = lambda i: (i, 0)
_rep = lambda i: (0, 0)

_k2 = pl.pallas_call(
    _k2_body,
    grid=(P // 128,),
    in_specs=[_blk((128, D), _row), _blk((D, D), _rep), _blk((128, 2), _row)],
    out_specs=[_blk((128, D), _row), _blk((128, 1), _row)],
    out_shape=(jax.ShapeDtypeStruct((P, D), jnp.float32),
               jax.ShapeDtypeStruct((P, 1), jnp.float32)),
)

_k4 = pl.pallas_call(
    _k4_body,
    grid=(P // 128,),
    in_specs=[_blk((128, D), _row), _blk((128, D), _row), _blk((128, D), _row),
              _blk((128, 1), _row), _blk((1, D), _rep), _blk((1, D), _rep),
              _blk((1, D), _rep), _blk((D, D), _rep)],
    out_specs=_blk((128, D), _row),
    out_shape=jax.ShapeDtypeStruct((P, D), jnp.float32),
)

_k5 = pl.pallas_call(
    _k5_body,
    grid=(P // 128,),
    in_specs=[_blk((128, D), _row), _blk((128, D), _row), _blk((128, D), _row),
              _blk((128, 1), _row), _blk((1, D), _rep)],
    out_specs=_blk((128, D), _row),
    out_shape=jax.ShapeDtypeStruct((P, D), jnp.float32),
)


def kernel(x, edge_index, edge_weight, W1, b1, ln_gamma, ln_beta, W2, b2):
    src = edge_index[0]
    dst = edge_index[1]
    w = edge_weight

    x_pad = jnp.zeros((P, D), jnp.float32).at[:N].set(x)

    deg2 = _k1(dst, w)
    degT = deg2.T
    h1, dinv = _k2(x_pad, W1, degT)
    dinv_flat = dinv.reshape(P)

    acc1 = _k3(h1, src, dst, w, dinv_flat)
    h2 = _k4(acc1[0], acc1[1], h1, dinv,
             b1.reshape(1, D), ln_gamma.reshape(1, D), ln_beta.reshape(1, D),
             W2)
    acc2 = _k3(h2, src, dst, w, dinv_flat)
    out = _k5(acc2[0], acc2[1], h2, dinv, b2.reshape(1, D))
    return out[:N]


# speedup vs baseline: 18.4729x; 1.9654x over previous
import functools

import jax
import jax.numpy as jnp
from jax import lax
from jax.experimental import pallas as pl
from jax.experimental.pallas import tpu as pltpu
from jax.experimental.pallas import tpu_sc as plsc

N = 10000
E = 320000
D = 128
P = 10240
NC = 2
NS = 16
WIN = 112
NWT = 93
EPT = NWT * WIN
E_PAD = EPT * NC * NS
REC = 3 * WIN
EPT3 = NWT * REC
ROWS_PER_TILE = P // NS
NBUF = 3

_mesh = plsc.VectorSubcoreMesh(core_axis_name="c", subcore_axis_name="s")
_sc_params = pltpu.CompilerParams(needs_layout_passes=False)


def _zero16():
    return jnp.zeros((16,), jnp.float32)


def _k1_body(ed_hbm, out_hbm, deg_sp, ew_all, dstw, ww, zero_v):
    c = lax.axis_index("c")
    s = lax.axis_index("s")
    wid = c * NS + s

    @pl.loop(0, ROWS_PER_TILE // 16)
    def _zero(j):
        zero_v[pl.ds(j * 16, 16)] = _zero16()

    pltpu.sync_copy(zero_v, deg_sp.at[pl.ds(s * ROWS_PER_TILE, ROWS_PER_TILE)])
    pltpu.sync_copy(ed_hbm.at[pl.ds(wid * EPT3, EPT3)], ew_all)
    plsc.subcore_barrier()

    @pl.loop(0, NWT)
    def _win(i):
        base = i * REC
        for f in range(WIN // 16):
            sl = pl.ds(f * 16, 16)
            dstw[sl] = ew_all[pl.ds(base + WIN + f * 16, 16)]
            ww[sl] = plsc.bitcast(ew_all[pl.ds(base + 2 * WIN + f * 16, 16)],
                                  jnp.float32)
        pltpu.sync_copy(ww, deg_sp.at[dstw], add=True)

    plsc.subcore_barrier()
    r0 = s * ROWS_PER_TILE
    pltpu.sync_copy(deg_sp.at[pl.ds(r0, ROWS_PER_TILE)],
                    out_hbm.at[c, pl.ds(r0, ROWS_PER_TILE)])


_k1 = pl.kernel(
    _k1_body,
    out_type=jax.ShapeDtypeStruct((NC, P), jnp.float32),
    mesh=_mesh,
    compiler_params=_sc_params,
    scratch_types=[
        pltpu.VMEM_SHARED((P,), jnp.float32),
        pltpu.VMEM((EPT3,), jnp.int32),
        pltpu.VMEM((WIN,), jnp.int32),
        pltpu.VMEM((WIN,), jnp.float32),
        pltpu.VMEM((ROWS_PER_TILE,), jnp.float32),
    ],
)


def _k3_body(h_hbm, ed_hbm, out_hbm,
             acc_sp, e0, e1, e2, r0_, r1_, r2_, d0, d1, d2, c0, c1, c2,
             ge0, ge1, ge2, gs0, gs1, gs2, se0, se1, se2):
    ew = (e0, e1, e2)
    rows = (r0_, r1_, r2_)
    dstw = (d0, d1, d2)
    cwin = (c0, c1, c2)
    gsem = (ge0, ge1, ge2)
    ssem = (gs0, gs1, gs2)
    esem = (se0, se1, se2)
    c = lax.axis_index("c")
    s = lax.axis_index("s")
    wid = c * NS + s
    ebase = wid * EPT3

    @pl.loop(0, 64)
    def _zero(j):
        for f in range(D // 16):
            r0_[j, pl.ds(f * 16, 16)] = _zero16()

    for t in range(ROWS_PER_TILE // 64):
        pltpu.sync_copy(r0_.at[pl.ds(0, 64)],
                        acc_sp.at[pl.ds(s * ROWS_PER_TILE + t * 64, 64)])
    plsc.subcore_barrier()

    for b in range(NBUF):
        pltpu.sync_copy(ed_hbm.at[pl.ds(ebase + b * REC, REC)], ew[b])
    for b in range(2):
        pltpu.async_copy(h_hbm.at[ew[b].at[pl.ds(0, WIN)]], rows[b], gsem[b])

    def _body(k, b):
        pltpu.make_async_copy(h_hbm.at[ew[b].at[pl.ds(0, WIN)]],
                              rows[b], gsem[b]).wait()

        for f in range(WIN // 16):
            sl = pl.ds(f * 16, 16)
            dstw[b][sl] = ew[b][pl.ds(WIN + f * 16, 16)]
            cwin[b][sl] = plsc.bitcast(ew[b][pl.ds(2 * WIN + f * 16, 16)],
                                       jnp.float32)

        @pl.when(k + NBUF < NWT)
        def _prefetch():
            pltpu.async_copy(ed_hbm.at[pl.ds(ebase + (k + NBUF) * REC, REC)],
                             ew[b], esem[b])

        @pl.loop(0, WIN, unroll=4)
        def _row(j):
            idx = jnp.broadcast_to(j, (16,)).astype(jnp.int32)
            cb = plsc.load_gather(cwin[b], [idx])
            for f in range(D // 16):
                sl = pl.ds(f * 16, 16)
                rows[b][j, sl] = rows[b][j, sl] * cb

        pltpu.async_copy(rows[b], acc_sp.at[dstw[b]], ssem[b], add=True)

        bn = (b + 2) % NBUF

        @pl.when(k >= 1)
        def _retire():
            pltpu.make_async_copy(rows[bn], acc_sp.at[dstw[bn]],
                                  ssem[bn]).wait()

        @pl.when(k + 2 < NWT)
        def _launch():
            @pl.when(k >= 1)
            def _ewwait():
                pltpu.make_async_copy(
                    ed_hbm.at[pl.ds(ebase + (k + 2) * REC, REC)],
                    ew[bn], esem[bn]).wait()

            pltpu.async_copy(h_hbm.at[ew[bn].at[pl.ds(0, WIN)]],
                             rows[bn], gsem[bn])

    @pl.loop(0, NWT // NBUF)
    def _outer(g):
        for b in range(NBUF):
            _body(g * NBUF + b, b)

    bl = (NWT - 1) % NBUF
    pltpu.make_async_copy(rows[bl], acc_sp.at[dstw[bl]], ssem[bl]).wait()

    plsc.subcore_barrier()
    for t in range(ROWS_PER_TILE // 128):
        rr = s * ROWS_PER_TILE + t * 128
        pltpu.sync_copy(acc_sp.at[pl.ds(rr, 128)], out_hbm.at[c, pl.ds(rr, 128)])


_k3 = pl.kernel(
    _k3_body,
    out_type=jax.ShapeDtypeStruct((NC, P, D), jnp.float32),
    mesh=_mesh,
    compiler_params=_sc_params,
    scratch_types=[
        pltpu.VMEM_SHARED((P, D), jnp.float32),
        pltpu.VMEM((REC,), jnp.int32),
        pltpu.VMEM((REC,), jnp.int32),
        pltpu.VMEM((REC,), jnp.int32),
        pltpu.VMEM((WIN, D), jnp.float32),
        pltpu.VMEM((WIN, D), jnp.float32),
        pltpu.VMEM((WIN, D), jnp.float32),
        pltpu.VMEM((WIN,), jnp.int32),
        pltpu.VMEM((WIN,), jnp.int32),
        pltpu.VMEM((WIN,), jnp.int32),
        pltpu.VMEM((WIN,), jnp.float32),
        pltpu.VMEM((WIN,), jnp.float32),
        pltpu.VMEM((WIN,), jnp.float32),
        pltpu.SemaphoreType.DMA,
        pltpu.SemaphoreType.DMA,
        pltpu.SemaphoreType.DMA,
        pltpu.SemaphoreType.DMA,
        pltpu.SemaphoreType.DMA,
        pltpu.SemaphoreType.DMA,
        pltpu.SemaphoreType.DMA,
        pltpu.SemaphoreType.DMA,
        pltpu.SemaphoreType.DMA,
    ],
)


def _k2a_body(x_ref, w1_ref, h_ref):
    h_ref[...] = jnp.dot(x_ref[...], w1_ref[...],
                         preferred_element_type=jnp.float32)


def _k2b_body(degT_ref, h1_ref, dinv_ref, h1s_ref):
    d = lax.rsqrt(degT_ref[:, 0:1] + degT_ref[:, 1:2] + 1.0)
    dinv_ref[...] = d
    h1s_ref[...] = d * h1_ref[...]


def _k4_body(a_ref, h1s_ref, dinv_ref, b1_ref, g_ref, b_ref, w2_ref, h2s_ref):
    d = dinv_ref[...]
    z = d * (a_ref[0] + a_ref[1] + h1s_ref[...]) + b1_ref[...]
    z = jnp.maximum(z, 0.0)
    m = jnp.mean(z, axis=-1, keepdims=True)
    zc = z - m
    v = jnp.mean(zc * zc, axis=-1, keepdims=True)
    zn = g_ref[...] * zc * lax.rsqrt(v + 1e-5) + b_ref[...]
    h2s_ref[...] = d * jnp.dot(zn, w2_ref[...],
                               preferred_element_type=jnp.float32)


def _k5_body(a_ref, h2s_ref, dinv_ref, b2_ref, o_ref):
    d = dinv_ref[...]
    o_ref[...] = d * (a_ref[0] + a_ref[1] + h2s_ref[...]) + b2_ref[...]


def _blk(shape, imap):
    return pl.BlockSpec(shape, imap)


_row = lambda i: (i, 0)
_rep = lambda i: (0, 0)
_acc = lambda i: (0, i, 0)

_k2a = pl.pallas_call(
    _k2a_body,
    grid=(P // 128,),
    in_specs=[_blk((128, D), _row), _blk((D, D), _rep)],
    out_specs=_blk((128, D), _row),
    out_shape=jax.ShapeDtypeStruct((P, D), jnp.float32),
)

_k2b = pl.pallas_call(
    _k2b_body,
    grid=(P // 128,),
    in_specs=[_blk((128, 2), _row), _blk((128, D), _row)],
    out_specs=[_blk((128, 1), _row), _blk((128, D), _row)],
    out_shape=(jax.ShapeDtypeStruct((P, 1), jnp.float32),
               jax.ShapeDtypeStruct((P, D), jnp.float32)),
)

_k4 = pl.pallas_call(
    _k4_body,
    grid=(P // 128,),
    in_specs=[_blk((NC, 128, D), _acc), _blk((128, D), _row),
              _blk((128, 1), _row), _blk((1, D), _rep), _blk((1, D), _rep),
              _blk((1, D), _rep), _blk((D, D), _rep)],
    out_specs=_blk((128, D), _row),
    out_shape=jax.ShapeDtypeStruct((P, D), jnp.float32),
)

_k5 = pl.pallas_call(
    _k5_body,
    grid=(P // 128,),
    in_specs=[_blk((NC, 128, D), _acc), _blk((128, D), _row),
              _blk((128, 1), _row), _blk((1, D), _rep)],
    out_specs=_blk((128, D), _row),
    out_shape=jax.ShapeDtypeStruct((P, D), jnp.float32),
)


def kernel(x, edge_index, edge_weight, W1, b1, ln_gamma, ln_beta, W2, b2):
    pad = E_PAD - E
    fill = (jnp.arange(pad, dtype=jnp.int32) * 37) % N
    src = jnp.concatenate([edge_index[0], fill])
    dst = jnp.concatenate([edge_index[1], fill])
    wb = lax.bitcast_convert_type(
        jnp.concatenate([edge_weight, jnp.zeros((pad,), jnp.float32)]),
        jnp.int32)
    edata = jnp.stack([src.reshape(-1, WIN), dst.reshape(-1, WIN),
                       wb.reshape(-1, WIN)], axis=1).reshape(-1)

    x_pad = jnp.zeros((P, D), jnp.float32).at[:N].set(x)

    deg2 = _k1(edata)
    h1 = _k2a(x_pad, W1)
    dinv, h1s = _k2b(deg2.T, h1)

    acc1 = _k3(h1s, edata)
    h2s = _k4(acc1, h1s, dinv,
              b1.reshape(1, D), ln_gamma.reshape(1, D), ln_beta.reshape(1, D),
              W2)
    acc2 = _k3(h2s, edata)
    out = _k5(acc2, h2s, dinv, b2.reshape(1, D))
    return out[:N]
